# Initial kernel scaffold; baseline (speedup 1.0000x reference)
import jax
import jax.numpy as jnp
from jax.experimental import pallas as pl


def kernel(x, edge_index, edge_attr, edge_sh, W_fc1, W_fc2, W0, b0, W1):
    raise NotImplementedError("write your pallas kernel here")


# speedup vs baseline: 1.0969x; 1.0969x over previous
import functools
import math

import jax
import jax.numpy as jnp
from jax import lax
from jax.experimental import pallas as pl
from jax.experimental.pallas import tpu as pltpu
from jax.experimental.pallas import tpu_sc as plsc

N_NODES = 10000
N_EDGES = 320000
D_FEAT = 128
D_EDGE = 16
HIDDEN = 32
N_CHUNKS = 4
CHUNK = 32
NSC = 16
EDGES_PER_TILE = N_EDGES // NSC
ROWS_PER_TILE = 624
W_WIN = 80
N_WIN = EDGES_PER_TILE // W_WIN

A0 = math.sqrt(1.0 / 2.0)
A1 = math.sqrt(3.0 / 2.0)
DENOM = math.sqrt(32.0)


def _mlp_body(ea_ref, sh_ref, wf1_ref, wf2_ref, wa_ref, w1_ref, sh16_ref):
    ea = ea_ref[...]
    h = jnp.dot(ea, wf1_ref[...], preferred_element_type=jnp.float32)
    h = jnp.maximum(h, 0.0) + jnp.log1p(jnp.exp(-jnp.abs(h))) - math.log(2.0)
    w = jnp.dot(h, wf2_ref[...], preferred_element_type=jnp.float32)
    sh0 = sh_ref[:, 0:1]
    wa = w[:, :128] * (A0 * sh0)
    w1 = w[:, 128:] * A1
    for k in range(N_CHUNKS):
        wa_ref[k] = wa[:, k * CHUNK:(k + 1) * CHUNK]
        w1_ref[k] = w1[:, k * CHUNK:(k + 1) * CHUNK]
    sh16_ref[...] = jnp.concatenate(
        [sh_ref[...], jnp.zeros((sh_ref.shape[0], 12), jnp.float32)], axis=1)


def _edge_mlp(edge_attr, edge_sh, wf1, wf2):
    be = 3200
    grid = (N_EDGES // be,)
    return pl.pallas_call(
        _mlp_body,
        grid=grid,
        in_specs=[
            pl.BlockSpec((be, D_EDGE), lambda i: (i, 0)),
            pl.BlockSpec((be, 4), lambda i: (i, 0)),
            pl.BlockSpec((D_EDGE, HIDDEN), lambda i: (0, 0)),
            pl.BlockSpec((HIDDEN, 256), lambda i: (0, 0)),
        ],
        out_specs=[
            pl.BlockSpec((N_CHUNKS, be, CHUNK), lambda i: (0, i, 0)),
            pl.BlockSpec((N_CHUNKS, be, CHUNK), lambda i: (0, i, 0)),
            pl.BlockSpec((be, 16), lambda i: (i, 0)),
        ],
        out_shape=[
            jax.ShapeDtypeStruct((N_CHUNKS, N_EDGES, CHUNK), jnp.float32),
            jax.ShapeDtypeStruct((N_CHUNKS, N_EDGES, CHUNK), jnp.float32),
            jax.ShapeDtypeStruct((N_EDGES, 16), jnp.float32),
        ],
    )(edge_attr, edge_sh, wf1, wf2)


def _sc_body(xf_hbm, src_hbm, dst_hbm, sh_hbm, waf_hbm, w1f_hbm, agg_hbm,
             srcbuf, idxbuf, dstbuf, wabuf, w1buf, shbuf, xbuf, updbuf, zbuf,
             acc, gsem):
    ci = lax.axis_index("c")
    si = lax.axis_index("s")
    ebase = si * EDGES_PER_TILE
    rbase = si * ROWS_PER_TILE

    def zrow(r, _):
        for g in range(8):
            zbuf[r, pl.ds(g * 16, 16)] = jnp.zeros((16,), jnp.float32)
        return _
    lax.fori_loop(0, 156, zrow, 0)

    for p in range(2):
        chunk = ci * 2 + p

        for j in range(4):
            pltpu.sync_copy(zbuf, acc.at[pl.ds(rbase + j * 156, 156)])
        @pl.when(si == NSC - 1)
        def _():
            pltpu.sync_copy(zbuf.at[pl.ds(0, 16)],
                            acc.at[pl.ds(NSC * ROWS_PER_TILE, 16)])
        plsc.subcore_barrier()

        def window(wi, _):
            row0 = ebase + wi * W_WIN
            pltpu.sync_copy(src_hbm.at[pl.ds(row0, W_WIN)], srcbuf)
            pltpu.sync_copy(dst_hbm.at[pl.ds(row0, W_WIN)], dstbuf)
            pltpu.sync_copy(sh_hbm.at[pl.ds(row0, W_WIN)], shbuf)
            pltpu.sync_copy(waf_hbm.at[pl.ds(chunk * N_EDGES + row0, W_WIN)],
                            wabuf)
            pltpu.sync_copy(w1f_hbm.at[pl.ds(chunk * N_EDGES + row0, W_WIN)],
                            w1buf)
            off = chunk * N_NODES
            for g in range(W_WIN // 16):
                idxbuf[pl.ds(g * 16, 16)] = srcbuf[pl.ds(g * 16, 16)] + off
            pltpu.async_copy(xf_hbm.at[idxbuf], xbuf, gsem).wait()

            def edge(i, _):
                shv = shbuf[i, :]
                sh1 = shv[1]
                sh2 = shv[2]
                sh3 = shv[3]
                for g in range(2):
                    xg = xbuf[i, pl.ds(g * 16, 16)]
                    m0 = wabuf[i, pl.ds(g * 16, 16)] * xg
                    t = w1buf[i, pl.ds(g * 16, 16)] * xg
                    updbuf[i, pl.ds(g * 16, 16)] = m0
                    updbuf[i, pl.ds(32 + g * 16, 16)] = t * sh1
                    updbuf[i, pl.ds(64 + g * 16, 16)] = t * sh2
                    updbuf[i, pl.ds(96 + g * 16, 16)] = t * sh3
                return _
            lax.fori_loop(0, W_WIN, edge, 0)

            pltpu.sync_copy(updbuf, acc.at[dstbuf], add=True)
            return _

        lax.fori_loop(0, N_WIN, window, 0)
        plsc.subcore_barrier()

        pltpu.sync_copy(acc.at[pl.ds(rbase, ROWS_PER_TILE)],
                        agg_hbm.at[pl.ds(chunk * N_NODES + rbase,
                                         ROWS_PER_TILE)])
        @pl.when(si == NSC - 1)
        def _():
            base = NSC * ROWS_PER_TILE
            pltpu.sync_copy(acc.at[pl.ds(base, 16)],
                            agg_hbm.at[pl.ds(chunk * N_NODES + base, 16)])
        plsc.subcore_barrier()


def _sc_aggregate(xflat, src, dst, edge_sh, waf, w1f):
    mesh = plsc.VectorSubcoreMesh(core_axis_name="c", subcore_axis_name="s")
    f = functools.partial(
        pl.kernel,
        out_type=jax.ShapeDtypeStruct((N_CHUNKS * N_NODES, 128), jnp.float32),
        mesh=mesh,
        scratch_types=[
            pltpu.VMEM((W_WIN,), jnp.int32),
            pltpu.VMEM((W_WIN,), jnp.int32),
            pltpu.VMEM((W_WIN,), jnp.int32),
            pltpu.VMEM((W_WIN, CHUNK), jnp.float32),
            pltpu.VMEM((W_WIN, CHUNK), jnp.float32),
            pltpu.VMEM((W_WIN, 16), jnp.float32),
            pltpu.VMEM((W_WIN, CHUNK), jnp.float32),
            pltpu.VMEM((W_WIN, 128), jnp.float32),
            pltpu.VMEM((156, 128), jnp.float32),
            pltpu.VMEM_SHARED((N_NODES, 128), jnp.float32),
            pltpu.SemaphoreType.DMA,
        ],
        compiler_params=pltpu.CompilerParams(use_tc_tiling_on_sc=False),
    )(_sc_body)
    return f(xflat, src, dst, edge_sh, waf, w1f)


def _out_body(agg_ref, w0_ref, b0_ref, w1_ref, perm_ref, out_ref):
    agg0 = jnp.concatenate([agg_ref[k, :, 0:CHUNK] for k in range(4)], axis=1)
    out0 = jnp.dot(agg0, w0_ref[...],
                   preferred_element_type=jnp.float32) + b0_ref[...]
    outs = []
    for m in range(3):
        a1m = jnp.concatenate(
            [agg_ref[k, :, (m + 1) * CHUNK:(m + 2) * CHUNK] for k in range(4)],
            axis=1)
        outs.append(jnp.dot(a1m, w1_ref[...],
                            preferred_element_type=jnp.float32))
    cat = jnp.concatenate(outs, axis=1)
    out1 = jnp.dot(cat, perm_ref[...],
                   preferred_element_type=jnp.float32)
    out_ref[...] = jnp.concatenate([out0, out1], axis=1)


def _out_linear(agg4, w0n, b0, w1n, perm):
    bn = 1000
    grid = (N_NODES // bn,)
    return pl.pallas_call(
        _out_body,
        grid=grid,
        in_specs=[
            pl.BlockSpec((N_CHUNKS, bn, 128), lambda i: (0, i, 0)),
            pl.BlockSpec((128, 128), lambda i: (0, 0)),
            pl.BlockSpec((128,), lambda i: (0,)),
            pl.BlockSpec((128, 64), lambda i: (0, 0)),
            pl.BlockSpec((192, 192), lambda i: (0, 0)),
        ],
        out_specs=pl.BlockSpec((bn, 320), lambda i: (i, 0)),
        out_shape=jax.ShapeDtypeStruct((N_NODES, 320), jnp.float32),
    )(agg4, w0n, b0, w1n, perm)


def kernel(x, edge_index, edge_attr, edge_sh, W_fc1, W_fc2, W0, b0, W1):
    src = edge_index[0]
    dst = edge_index[1]
    wf1 = W_fc1 / math.sqrt(float(D_EDGE))
    wf2 = W_fc2 / math.sqrt(float(HIDDEN))

    wa4, w14, sh16 = _edge_mlp(edge_attr, edge_sh, wf1, wf2)
    waf = wa4.reshape(N_CHUNKS * N_EDGES, CHUNK)
    w1f = w14.reshape(N_CHUNKS * N_EDGES, CHUNK)

    xflat = x.reshape(N_NODES, N_CHUNKS, CHUNK).transpose(1, 0, 2) \
             .reshape(N_CHUNKS * N_NODES, CHUNK)

    aggf = _sc_aggregate(xflat, src, dst, sh16, waf, w1f)
    agg4 = aggf.reshape(N_CHUNKS, N_NODES, 128)

    w0n = W0 / (math.sqrt(128.0) * DENOM)
    w1n = W1 / (math.sqrt(128.0) * DENOM)
    perm = jnp.zeros((192, 192), jnp.float32)
    idx_src = jnp.arange(192)
    m = idx_src // 64
    o = idx_src % 64
    perm = perm.at[idx_src, 3 * o + m].set(1.0)

    return _out_linear(agg4, w0n, b0, w1n, perm)


# speedup vs baseline: 1.8892x; 1.7223x over previous
import functools
import math

import jax
import jax.numpy as jnp
from jax import lax
from jax.experimental import pallas as pl
from jax.experimental.pallas import tpu as pltpu
from jax.experimental.pallas import tpu_sc as plsc

N_NODES = 10000
N_EDGES = 320000
D_FEAT = 128
D_EDGE = 16
HIDDEN = 32
N_CHUNKS = 4
CHUNK = 32
RECW = 80
NSC = 16
EDGES_PER_TILE = N_EDGES // NSC
ROWS_PER_TILE = 624
W_WIN = 80
N_WIN = EDGES_PER_TILE // W_WIN
SW_EDGES = 4000
SW_WINS = SW_EDGES // W_WIN
N_SW = EDGES_PER_TILE // SW_EDGES

A0 = math.sqrt(1.0 / 2.0)
A1 = math.sqrt(3.0 / 2.0)
DENOM = math.sqrt(32.0)


def _mlp_body(ea_ref, sh_ref, wf1_ref, wf2_ref, rec_ref):
    ea = ea_ref[...]
    h = jnp.dot(ea, wf1_ref[...], preferred_element_type=jnp.float32)
    h = jnp.maximum(h, 0.0) + jnp.log1p(jnp.exp(-jnp.abs(h))) - math.log(2.0)
    w = jnp.dot(h, wf2_ref[...], preferred_element_type=jnp.float32)
    sh0 = sh_ref[:, 0:1]
    wa = w[:, :128] * (A0 * sh0)
    w1 = w[:, 128:] * A1
    be = ea.shape[0]
    shp = jnp.concatenate(
        [sh_ref[:, 1:4], jnp.zeros((be, 13), jnp.float32)], axis=1)
    for k in range(N_CHUNKS):
        rec_ref[k, :, 0:CHUNK] = wa[:, k * CHUNK:(k + 1) * CHUNK]
        rec_ref[k, :, CHUNK:2 * CHUNK] = w1[:, k * CHUNK:(k + 1) * CHUNK]
        rec_ref[k, :, 64:80] = shp


def _edge_mlp(edge_attr, edge_sh, wf1, wf2):
    be = 3200
    grid = (N_EDGES // be,)
    return pl.pallas_call(
        _mlp_body,
        grid=grid,
        in_specs=[
            pl.BlockSpec((be, D_EDGE), lambda i: (i, 0)),
            pl.BlockSpec((be, 4), lambda i: (i, 0)),
            pl.BlockSpec((D_EDGE, HIDDEN), lambda i: (0, 0)),
            pl.BlockSpec((HIDDEN, 256), lambda i: (0, 0)),
        ],
        out_specs=pl.BlockSpec((N_CHUNKS, be, RECW), lambda i: (0, i, 0)),
        out_shape=jax.ShapeDtypeStruct((N_CHUNKS, N_EDGES, RECW),
                                       jnp.float32),
    )(edge_attr, edge_sh, wf1, wf2)


def _sc_body(xf_hbm, src_hbm, dst_hbm, recf_hbm, agg_hbm,
             srcsw, dstsw, rec0, rec1, x0, x1, upd0, upd1,
             dstb0, dstb1, zbuf, acc,
             rsem0, rsem1, gsem0, gsem1, ssem0, ssem1):
    ci = lax.axis_index("c")
    si = lax.axis_index("s")
    ebase = si * EDGES_PER_TILE
    rbase = si * ROWS_PER_TILE

    recs = (rec0, rec1)
    xs = (x0, x1)
    upds = (upd0, upd1)
    dstbs = (dstb0, dstb1)
    rsems = (rsem0, rsem1)
    gsems = (gsem0, gsem1)
    ssems = (ssem0, ssem1)

    def zrow(r, _):
        for g in range(8):
            zbuf[r, pl.ds(g * 16, 16)] = jnp.zeros((16,), jnp.float32)
        return _
    lax.fori_loop(0, 16, zrow, 0)

    for p in range(2):
        chunk = ci * 2 + p
        off = chunk * N_NODES
        recbase = chunk * N_EDGES + ebase

        def zacc(j, _):
            pltpu.sync_copy(zbuf, acc.at[pl.ds(rbase + j * 16, 16)])
            return _
        lax.fori_loop(0, ROWS_PER_TILE // 16, zacc, 0)
        @pl.when(si == NSC - 1)
        def _():
            pltpu.sync_copy(zbuf, acc.at[pl.ds(NSC * ROWS_PER_TILE, 16)])
        plsc.subcore_barrier()

        def rec_issue(w, u):
            pltpu.async_copy(
                recf_hbm.at[pl.ds(recbase + w * W_WIN, W_WIN)],
                recs[u], rsems[u])

        def rec_wait(w, u):
            pltpu.make_async_copy(
                recf_hbm.at[pl.ds(recbase + w * W_WIN, W_WIN)],
                recs[u], rsems[u]).wait()

        def gather_issue(k, u):
            pltpu.async_copy(
                xf_hbm.at[srcsw.at[pl.ds(k * W_WIN, W_WIN)]],
                xs[u], gsems[u])

        def gather_wait(k, u):
            pltpu.make_async_copy(
                xf_hbm.at[srcsw.at[pl.ds(k * W_WIN, W_WIN)]],
                xs[u], gsems[u]).wait()

        def scatter_wait(u):
            pltpu.make_async_copy(upds[u], acc.at[dstbs[u]],
                                  ssems[u]).wait()

        rec_issue(0, 0)
        rec_issue(1, 1)

        for sw in range(N_SW):
            wb = sw * SW_WINS
            pltpu.sync_copy(src_hbm.at[pl.ds(ebase + sw * SW_EDGES,
                                             SW_EDGES)], srcsw)
            pltpu.sync_copy(dst_hbm.at[pl.ds(ebase + sw * SW_EDGES,
                                             SW_EDGES)], dstsw)

            def addoff(g, _):
                srcsw[pl.ds(g * 16, 16)] = srcsw[pl.ds(g * 16, 16)] + off
                return _
            lax.fori_loop(0, SW_EDGES // 16, addoff, 0)

            def window(j, k, u, first):
                w = wb + k
                rec_b, x_b, upd_b, dstb_b = recs[u], xs[u], upds[u], dstbs[u]
                nu = 1 - u

                if first:
                    @pl.when(j == 0)
                    def _():
                        rec_wait(wb, u)
                        gather_issue(0, u)
                gather_wait(k, u)
                if sw == 0:
                    @pl.when(w >= 2)
                    def _():
                        scatter_wait(u)
                else:
                    scatter_wait(u)

                def bdst(g, _):
                    dstb_b[pl.ds(g * 16, 16)] = dstsw[
                        pl.ds(k * W_WIN + g * 16, 16)]
                    return _
                lax.fori_loop(0, W_WIN // 16, bdst, 0)

                def edge(i, _):
                    shv = rec_b[i, pl.ds(64, 16)]
                    sh1 = shv[0]
                    sh2 = shv[1]
                    sh3 = shv[2]
                    for g in range(2):
                        xg = x_b[i, pl.ds(g * 16, 16)]
                        m0 = rec_b[i, pl.ds(g * 16, 16)] * xg
                        t = rec_b[i, pl.ds(CHUNK + g * 16, 16)] * xg
                        upd_b[i, pl.ds(g * 16, 16)] = m0
                        upd_b[i, pl.ds(32 + g * 16, 16)] = t * sh1
                        upd_b[i, pl.ds(64 + g * 16, 16)] = t * sh2
                        upd_b[i, pl.ds(96 + g * 16, 16)] = t * sh3
                    return _
                lax.fori_loop(0, W_WIN, edge, 0)

                pltpu.async_copy(upd_b, acc.at[dstb_b], ssems[u], add=True)

                if sw == N_SW - 1:
                    @pl.when(w < N_WIN - 2)
                    def _():
                        rec_issue(w + 2, u)
                else:
                    rec_issue(w + 2, u)

                def prep(_k):
                    rec_wait(w + 1, nu)
                    gather_issue(_k, nu)
                if u == 0:
                    prep(k + 1)
                else:
                    @pl.when(j < SW_WINS // 2 - 1)
                    def _():
                        prep(k + 1)

            def body2(j, _):
                window(j, 2 * j, 0, True)
                window(j, 2 * j + 1, 1, False)
                return _
            lax.fori_loop(0, SW_WINS // 2, body2, 0)

        scatter_wait(0)
        scatter_wait(1)
        plsc.subcore_barrier()

        pltpu.sync_copy(acc.at[pl.ds(rbase, ROWS_PER_TILE)],
                        agg_hbm.at[pl.ds(chunk * N_NODES + rbase,
                                         ROWS_PER_TILE)])
        @pl.when(si == NSC - 1)
        def _():
            base = NSC * ROWS_PER_TILE
            pltpu.sync_copy(acc.at[pl.ds(base, 16)],
                            agg_hbm.at[pl.ds(chunk * N_NODES + base, 16)])
        plsc.subcore_barrier()


def _sc_aggregate(xflat, src, dst, recf):
    mesh = plsc.VectorSubcoreMesh(core_axis_name="c", subcore_axis_name="s")
    f = functools.partial(
        pl.kernel,
        out_type=jax.ShapeDtypeStruct((N_CHUNKS * N_NODES, 128), jnp.float32),
        mesh=mesh,
        scratch_types=[
            pltpu.VMEM((SW_EDGES,), jnp.int32),
            pltpu.VMEM((SW_EDGES,), jnp.int32),
            pltpu.VMEM((W_WIN, RECW), jnp.float32),
            pltpu.VMEM((W_WIN, RECW), jnp.float32),
            pltpu.VMEM((W_WIN, CHUNK), jnp.float32),
            pltpu.VMEM((W_WIN, CHUNK), jnp.float32),
            pltpu.VMEM((W_WIN, 128), jnp.float32),
            pltpu.VMEM((W_WIN, 128), jnp.float32),
            pltpu.VMEM((W_WIN,), jnp.int32),
            pltpu.VMEM((W_WIN,), jnp.int32),
            pltpu.VMEM((16, 128), jnp.float32),
            pltpu.VMEM_SHARED((N_NODES, 128), jnp.float32),
            pltpu.SemaphoreType.DMA,
            pltpu.SemaphoreType.DMA,
            pltpu.SemaphoreType.DMA,
            pltpu.SemaphoreType.DMA,
            pltpu.SemaphoreType.DMA,
            pltpu.SemaphoreType.DMA,
        ],
        compiler_params=pltpu.CompilerParams(use_tc_tiling_on_sc=False),
    )(_sc_body)
    return f(xflat, src, dst, recf)


def _out_body(agg_ref, w0_ref, b0_ref, w1_ref, perm_ref, out_ref):
    agg0 = jnp.concatenate([agg_ref[k, :, 0:CHUNK] for k in range(4)], axis=1)
    out0 = jnp.dot(agg0, w0_ref[...],
                   preferred_element_type=jnp.float32) + b0_ref[...]
    outs = []
    for m in range(3):
        a1m = jnp.concatenate(
            [agg_ref[k, :, (m + 1) * CHUNK:(m + 2) * CHUNK] for k in range(4)],
            axis=1)
        outs.append(jnp.dot(a1m, w1_ref[...],
                            preferred_element_type=jnp.float32))
    cat = jnp.concatenate(outs, axis=1)
    out1 = jnp.dot(cat, perm_ref[...],
                   preferred_element_type=jnp.float32,
                   precision=lax.Precision.HIGHEST)
    out_ref[...] = jnp.concatenate([out0, out1], axis=1)


def _out_linear(agg4, w0n, b0, w1n, perm):
    bn = 1000
    grid = (N_NODES // bn,)
    return pl.pallas_call(
        _out_body,
        grid=grid,
        in_specs=[
            pl.BlockSpec((N_CHUNKS, bn, 128), lambda i: (0, i, 0)),
            pl.BlockSpec((128, 128), lambda i: (0, 0)),
            pl.BlockSpec((128,), lambda i: (0,)),
            pl.BlockSpec((128, 64), lambda i: (0, 0)),
            pl.BlockSpec((192, 192), lambda i: (0, 0)),
        ],
        out_specs=pl.BlockSpec((bn, 320), lambda i: (i, 0)),
        out_shape=jax.ShapeDtypeStruct((N_NODES, 320), jnp.float32),
    )(agg4, w0n, b0, w1n, perm)


def kernel(x, edge_index, edge_attr, edge_sh, W_fc1, W_fc2, W0, b0, W1):
    import numpy as np
    src = edge_index[0]
    dst = edge_index[1]
    wf1 = W_fc1 / math.sqrt(float(D_EDGE))
    wf2 = W_fc2 / math.sqrt(float(HIDDEN))

    rec4 = _edge_mlp(edge_attr, edge_sh, wf1, wf2)
    recf = rec4.reshape(N_CHUNKS * N_EDGES, RECW)

    xflat = x.reshape(N_NODES, N_CHUNKS, CHUNK).transpose(1, 0, 2) \
             .reshape(N_CHUNKS * N_NODES, CHUNK)

    aggf = _sc_aggregate(xflat, src, dst, recf)
    agg4 = aggf.reshape(N_CHUNKS, N_NODES, 128)

    w0n = W0 / (math.sqrt(128.0) * DENOM)
    w1n = W1 / (math.sqrt(128.0) * DENOM)
    pm = np.zeros((192, 192), np.float32)
    i_src = np.arange(192)
    pm[i_src, 3 * (i_src % 64) + i_src // 64] = 1.0
    perm = jnp.asarray(pm)

    return _out_linear(agg4, w0n, b0, w1n, perm)


# speedup vs baseline: 2.6652x; 1.4107x over previous
import functools
import math

import jax
import jax.numpy as jnp
from jax import lax
from jax.experimental import pallas as pl
from jax.experimental.pallas import tpu as pltpu
from jax.experimental.pallas import tpu_sc as plsc

N_NODES = 10000
N_EDGES = 320000
D_FEAT = 128
D_EDGE = 16
HIDDEN = 32
N_CHUNKS = 4
CHUNK = 32
RECW = 128
RECU = 80
NSC = 16
EDGES_PER_TILE = N_EDGES // NSC
ROWS_PER_TILE = 624
W_WIN = 80
N_WIN = EDGES_PER_TILE // W_WIN
SW_EDGES = 4000
SW_WINS = SW_EDGES // W_WIN
N_SW = EDGES_PER_TILE // SW_EDGES

A0 = math.sqrt(1.0 / 2.0)
A1 = math.sqrt(3.0 / 2.0)
DENOM = math.sqrt(32.0)


def _mlp_body(ea_ref, sh_ref, wf1_ref, wf2_ref, rec_ref):
    ea = ea_ref[...]
    h = jnp.dot(ea, wf1_ref[...], preferred_element_type=jnp.float32,
                precision=lax.Precision.HIGHEST)
    h = jnp.maximum(h, 0.0) + jnp.log1p(jnp.exp(-jnp.abs(h))) - math.log(2.0)
    w = jnp.dot(h, wf2_ref[...], preferred_element_type=jnp.float32,
                precision=lax.Precision.HIGHEST)
    sh0 = sh_ref[:, 0:1]
    wa = w[:, :128] * (A0 * sh0)
    w1 = w[:, 128:] * A1
    be = ea.shape[0]
    shp = jnp.concatenate(
        [sh_ref[:, 1:4], jnp.zeros((be, 13), jnp.float32)], axis=1)
    for k in range(N_CHUNKS):
        rec_ref[k, :, 0:CHUNK] = wa[:, k * CHUNK:(k + 1) * CHUNK]
        rec_ref[k, :, CHUNK:2 * CHUNK] = w1[:, k * CHUNK:(k + 1) * CHUNK]
        rec_ref[k, :, 64:80] = shp


def _edge_mlp(edge_attr, edge_sh, wf1, wf2):
    be = 3200
    grid = (N_EDGES // be,)
    return pl.pallas_call(
        _mlp_body,
        grid=grid,
        in_specs=[
            pl.BlockSpec((be, D_EDGE), lambda i: (i, 0)),
            pl.BlockSpec((be, 4), lambda i: (i, 0)),
            pl.BlockSpec((D_EDGE, HIDDEN), lambda i: (0, 0)),
            pl.BlockSpec((HIDDEN, 256), lambda i: (0, 0)),
        ],
        out_specs=pl.BlockSpec((N_CHUNKS, be, RECW), lambda i: (0, i, 0)),
        out_shape=jax.ShapeDtypeStruct((N_CHUNKS, N_EDGES, RECW),
                                       jnp.float32),
    )(edge_attr, edge_sh, wf1, wf2)


def _sc_body(xf_hbm, src_hbm, dst_hbm, recf_hbm, agg_hbm,
             srcsw, dstsw, rec0, rec1, x0, x1, upd0, upd1,
             dstb0, dstb1, zbuf, acc,
             rsem0, rsem1, gsem0, gsem1, ssem0, ssem1):
    ci = lax.axis_index("c")
    si = lax.axis_index("s")
    ebase = si * EDGES_PER_TILE
    rbase = si * ROWS_PER_TILE

    recs = (rec0, rec1)
    xs = (x0, x1)
    upds = (upd0, upd1)
    dstbs = (dstb0, dstb1)
    rsems = (rsem0, rsem1)
    gsems = (gsem0, gsem1)
    ssems = (ssem0, ssem1)

    def zrow(r, _):
        for g in range(8):
            zbuf[r, pl.ds(g * 16, 16)] = jnp.zeros((16,), jnp.float32)
        return _
    lax.fori_loop(0, 16, zrow, 0)

    for p in range(2):
        chunk = ci * 2 + p
        off = chunk * N_NODES
        recbase = chunk * N_EDGES + ebase

        def zacc(j, _):
            pltpu.sync_copy(zbuf, acc.at[pl.ds(rbase + j * 16, 16)])
            return _
        lax.fori_loop(0, ROWS_PER_TILE // 16, zacc, 0)
        @pl.when(si == NSC - 1)
        def _():
            pltpu.sync_copy(zbuf, acc.at[pl.ds(NSC * ROWS_PER_TILE, 16)])
        plsc.subcore_barrier()

        def rec_issue(w, u):
            pltpu.async_copy(
                recf_hbm.at[pl.ds(recbase + w * W_WIN, W_WIN),
                            pl.ds(0, RECU)],
                recs[u], rsems[u])

        def rec_wait(w, u):
            pltpu.make_async_copy(
                recf_hbm.at[pl.ds(recbase + w * W_WIN, W_WIN),
                            pl.ds(0, RECU)],
                recs[u], rsems[u]).wait()

        def gather_issue(k, u):
            pltpu.async_copy(
                xf_hbm.at[srcsw.at[pl.ds(k * W_WIN, W_WIN)]],
                xs[u], gsems[u])

        def gather_wait(k, u):
            pltpu.make_async_copy(
                xf_hbm.at[srcsw.at[pl.ds(k * W_WIN, W_WIN)]],
                xs[u], gsems[u]).wait()

        def scatter_wait(u):
            pltpu.make_async_copy(upds[u], acc.at[dstbs[u]],
                                  ssems[u]).wait()

        rec_issue(0, 0)
        rec_issue(1, 1)

        for sw in range(N_SW):
            wb = sw * SW_WINS
            pltpu.sync_copy(src_hbm.at[pl.ds(ebase + sw * SW_EDGES,
                                             SW_EDGES)], srcsw)
            pltpu.sync_copy(dst_hbm.at[pl.ds(ebase + sw * SW_EDGES,
                                             SW_EDGES)], dstsw)

            def addoff(g, _):
                srcsw[pl.ds(g * 16, 16)] = srcsw[pl.ds(g * 16, 16)] + off
                return _
            lax.fori_loop(0, SW_EDGES // 16, addoff, 0)

            def window(j, k, u, first):
                w = wb + k
                rec_b, x_b, upd_b, dstb_b = recs[u], xs[u], upds[u], dstbs[u]
                nu = 1 - u

                if first:
                    @pl.when(j == 0)
                    def _():
                        rec_wait(wb, u)
                        gather_issue(0, u)
                gather_wait(k, u)
                if sw == 0:
                    @pl.when(w >= 2)
                    def _():
                        scatter_wait(u)
                else:
                    scatter_wait(u)

                def bdst(g, _):
                    dstb_b[pl.ds(g * 16, 16)] = dstsw[
                        pl.ds(k * W_WIN + g * 16, 16)]
                    return _
                lax.fori_loop(0, W_WIN // 16, bdst, 0)

                @plsc.parallel_loop(0, W_WIN, 1, unroll=4)
                def edge(i):
                    shv = rec_b[i, pl.ds(64, 16)]
                    sh1 = shv[0]
                    sh2 = shv[1]
                    sh3 = shv[2]
                    for g in range(2):
                        xg = x_b[i, pl.ds(g * 16, 16)]
                        m0 = rec_b[i, pl.ds(g * 16, 16)] * xg
                        t = rec_b[i, pl.ds(CHUNK + g * 16, 16)] * xg
                        upd_b[i, pl.ds(g * 16, 16)] = m0
                        upd_b[i, pl.ds(32 + g * 16, 16)] = t * sh1
                        upd_b[i, pl.ds(64 + g * 16, 16)] = t * sh2
                        upd_b[i, pl.ds(96 + g * 16, 16)] = t * sh3

                pltpu.async_copy(upd_b, acc.at[dstb_b], ssems[u], add=True)

                if sw == N_SW - 1:
                    @pl.when(w < N_WIN - 2)
                    def _():
                        rec_issue(w + 2, u)
                else:
                    rec_issue(w + 2, u)

                def prep(_k):
                    rec_wait(w + 1, nu)
                    gather_issue(_k, nu)
                if u == 0:
                    prep(k + 1)
                else:
                    @pl.when(j < SW_WINS // 2 - 1)
                    def _():
                        prep(k + 1)

            def body2(j, _):
                window(j, 2 * j, 0, True)
                window(j, 2 * j + 1, 1, False)
                return _
            lax.fori_loop(0, SW_WINS // 2, body2, 0)

        scatter_wait(0)
        scatter_wait(1)
        plsc.subcore_barrier()

        pltpu.sync_copy(acc.at[pl.ds(rbase, ROWS_PER_TILE)],
                        agg_hbm.at[pl.ds(chunk * N_NODES + rbase,
                                         ROWS_PER_TILE)])
        @pl.when(si == NSC - 1)
        def _():
            base = NSC * ROWS_PER_TILE
            pltpu.sync_copy(acc.at[pl.ds(base, 16)],
                            agg_hbm.at[pl.ds(chunk * N_NODES + base, 16)])
        plsc.subcore_barrier()


def _sc_aggregate(xflat, src, dst, recf):
    mesh = plsc.VectorSubcoreMesh(core_axis_name="c", subcore_axis_name="s")
    f = functools.partial(
        pl.kernel,
        out_type=jax.ShapeDtypeStruct((N_CHUNKS * N_NODES, 128), jnp.float32),
        mesh=mesh,
        scratch_types=[
            pltpu.VMEM((SW_EDGES,), jnp.int32),
            pltpu.VMEM((SW_EDGES,), jnp.int32),
            pltpu.VMEM((W_WIN, RECU), jnp.float32),
            pltpu.VMEM((W_WIN, RECU), jnp.float32),
            pltpu.VMEM((W_WIN, CHUNK), jnp.float32),
            pltpu.VMEM((W_WIN, CHUNK), jnp.float32),
            pltpu.VMEM((W_WIN, 128), jnp.float32),
            pltpu.VMEM((W_WIN, 128), jnp.float32),
            pltpu.VMEM((W_WIN,), jnp.int32),
            pltpu.VMEM((W_WIN,), jnp.int32),
            pltpu.VMEM((16, 128), jnp.float32),
            pltpu.VMEM_SHARED((N_NODES, 128), jnp.float32),
            pltpu.SemaphoreType.DMA,
            pltpu.SemaphoreType.DMA,
            pltpu.SemaphoreType.DMA,
            pltpu.SemaphoreType.DMA,
            pltpu.SemaphoreType.DMA,
            pltpu.SemaphoreType.DMA,
        ],
        compiler_params=pltpu.CompilerParams(use_tc_tiling_on_sc=False),
    )(_sc_body)
    return f(xflat, src, dst, recf)


def _out_body(agg_ref, w0_ref, b0_ref, w1_ref, perm_ref, out_ref):
    agg0 = jnp.concatenate([agg_ref[k, :, 0:CHUNK] for k in range(4)], axis=1)
    out0 = jnp.dot(agg0, w0_ref[...],
                   preferred_element_type=jnp.float32) + b0_ref[...]
    outs = []
    for m in range(3):
        a1m = jnp.concatenate(
            [agg_ref[k, :, (m + 1) * CHUNK:(m + 2) * CHUNK] for k in range(4)],
            axis=1)
        outs.append(jnp.dot(a1m, w1_ref[...],
                            preferred_element_type=jnp.float32))
    cat = jnp.concatenate(outs, axis=1)
    out1 = jnp.dot(cat, perm_ref[...],
                   preferred_element_type=jnp.float32,
                   precision=lax.Precision.HIGHEST)
    out_ref[...] = jnp.concatenate([out0, out1], axis=1)


def _out_linear(agg4, w0n, b0, w1n, perm):
    bn = 1000
    grid = (N_NODES // bn,)
    return pl.pallas_call(
        _out_body,
        grid=grid,
        in_specs=[
            pl.BlockSpec((N_CHUNKS, bn, 128), lambda i: (0, i, 0)),
            pl.BlockSpec((128, 128), lambda i: (0, 0)),
            pl.BlockSpec((128,), lambda i: (0,)),
            pl.BlockSpec((128, 64), lambda i: (0, 0)),
            pl.BlockSpec((192, 192), lambda i: (0, 0)),
        ],
        out_specs=pl.BlockSpec((bn, 320), lambda i: (i, 0)),
        out_shape=jax.ShapeDtypeStruct((N_NODES, 320), jnp.float32),
    )(agg4, w0n, b0, w1n, perm)


def kernel(x, edge_index, edge_attr, edge_sh, W_fc1, W_fc2, W0, b0, W1):
    import numpy as np
    src = edge_index[0]
    dst = edge_index[1]
    wf1 = W_fc1 / math.sqrt(float(D_EDGE))
    wf2 = W_fc2 / math.sqrt(float(HIDDEN))

    rec4 = _edge_mlp(edge_attr, edge_sh, wf1, wf2)
    recf = rec4.reshape(N_CHUNKS * N_EDGES, RECW)

    xflat = x.reshape(N_NODES, N_CHUNKS, CHUNK).transpose(1, 0, 2) \
             .reshape(N_CHUNKS * N_NODES, CHUNK)

    aggf = _sc_aggregate(xflat, src, dst, recf)
    agg4 = aggf.reshape(N_CHUNKS, N_NODES, 128)

    w0n = W0 / (math.sqrt(128.0) * DENOM)
    w1n = W1 / (math.sqrt(128.0) * DENOM)
    pm = np.zeros((192, 192), np.float32)
    i_src = np.arange(192)
    pm[i_src, 3 * (i_src % 64) + i_src // 64] = 1.0
    perm = jnp.asarray(pm)

    return _out_linear(agg4, w0n, b0, w1n, perm)


# speedup vs baseline: 3.5410x; 1.3286x over previous
import functools
import math

import jax
import jax.numpy as jnp
from jax import lax
from jax.experimental import pallas as pl
from jax.experimental.pallas import tpu as pltpu
from jax.experimental.pallas import tpu_sc as plsc

N_NODES = 10000
N_EDGES = 320000
D_FEAT = 128
D_EDGE = 16
HIDDEN = 32
N_CHUNKS = 4
CHUNK = 32
RECW = 128
RECU = 80
NSC = 16
EDGES_PER_TILE = N_EDGES // NSC
ROWS_PER_TILE = 624
W_WIN = 80
N_WIN = EDGES_PER_TILE // W_WIN
SW_EDGES = 4000
SW_WINS = SW_EDGES // W_WIN
N_SW = EDGES_PER_TILE // SW_EDGES

A0 = math.sqrt(1.0 / 2.0)
A1 = math.sqrt(3.0 / 2.0)
DENOM = math.sqrt(32.0)


def _mlp_body(ea_ref, sh_ref, wf1_ref, wf2_ref, rec_ref):
    ea = ea_ref[...]
    h = jnp.dot(ea, wf1_ref[...], preferred_element_type=jnp.float32)
    h = jnp.maximum(h, 0.0) + jnp.log1p(jnp.exp(-jnp.abs(h))) - math.log(2.0)
    w = jnp.dot(h, wf2_ref[...], preferred_element_type=jnp.float32)
    sh0 = sh_ref[:, 0:1]
    wa = w[:, :128] * (A0 * sh0)
    w1 = w[:, 128:] * A1
    be = ea.shape[0]
    shp = jnp.concatenate(
        [sh_ref[:, 1:4], jnp.zeros((be, 13), jnp.float32)], axis=1)
    for k in range(N_CHUNKS):
        rec_ref[k, :, 0:CHUNK] = wa[:, k * CHUNK:(k + 1) * CHUNK]
        rec_ref[k, :, CHUNK:2 * CHUNK] = w1[:, k * CHUNK:(k + 1) * CHUNK]
        rec_ref[k, :, 64:80] = shp


def _edge_mlp(edge_attr, edge_sh, wf1, wf2):
    be = 3200
    grid = (N_EDGES // be,)
    return pl.pallas_call(
        _mlp_body,
        grid=grid,
        in_specs=[
            pl.BlockSpec((be, D_EDGE), lambda i: (i, 0)),
            pl.BlockSpec((be, 4), lambda i: (i, 0)),
            pl.BlockSpec((D_EDGE, HIDDEN), lambda i: (0, 0)),
            pl.BlockSpec((HIDDEN, 256), lambda i: (0, 0)),
        ],
        out_specs=pl.BlockSpec((N_CHUNKS, be, RECW), lambda i: (0, i, 0)),
        out_shape=jax.ShapeDtypeStruct((N_CHUNKS, N_EDGES, RECW),
                                       jnp.float32),
    )(edge_attr, edge_sh, wf1, wf2)


def _sc_body(xf_hbm, src_hbm, dst_hbm, recf_hbm, agg_hbm,
             srcsw, dstsw, rec0, rec1, x0, x1, upd0, upd1,
             dstb0, dstb1, zbuf, acc,
             rsem0, rsem1, gsem0, gsem1, ssem0, ssem1):
    ci = lax.axis_index("c")
    si = lax.axis_index("s")
    ebase = si * EDGES_PER_TILE
    rbase = si * ROWS_PER_TILE

    recs = (rec0, rec1)
    xs = (x0, x1)
    upds = (upd0, upd1)
    dstbs = (dstb0, dstb1)
    rsems = (rsem0, rsem1)
    gsems = (gsem0, gsem1)
    ssems = (ssem0, ssem1)

    def zrow(r, _):
        for g in range(8):
            zbuf[r, pl.ds(g * 16, 16)] = jnp.zeros((16,), jnp.float32)
        return _
    lax.fori_loop(0, 16, zrow, 0)

    for p in range(2):
        chunk = ci * 2 + p
        off = chunk * N_NODES
        recbase = chunk * N_EDGES + ebase

        def zacc(j, _):
            pltpu.sync_copy(zbuf, acc.at[pl.ds(rbase + j * 16, 16)])
            return _
        lax.fori_loop(0, ROWS_PER_TILE // 16, zacc, 0)
        @pl.when(si == NSC - 1)
        def _():
            pltpu.sync_copy(zbuf, acc.at[pl.ds(NSC * ROWS_PER_TILE, 16)])
        plsc.subcore_barrier()

        def rec_issue(w, u):
            pltpu.async_copy(
                recf_hbm.at[pl.ds(recbase + w * W_WIN, W_WIN),
                            pl.ds(0, RECU)],
                recs[u], rsems[u])

        def rec_wait(w, u):
            pltpu.make_async_copy(
                recf_hbm.at[pl.ds(recbase + w * W_WIN, W_WIN),
                            pl.ds(0, RECU)],
                recs[u], rsems[u]).wait()

        def gather_issue(k, u):
            pltpu.async_copy(
                xf_hbm.at[srcsw.at[pl.ds(k * W_WIN, W_WIN)]],
                xs[u], gsems[u])

        def gather_wait(k, u):
            pltpu.make_async_copy(
                xf_hbm.at[srcsw.at[pl.ds(k * W_WIN, W_WIN)]],
                xs[u], gsems[u]).wait()

        def scatter_wait(u):
            pltpu.make_async_copy(upds[u], acc.at[dstbs[u]],
                                  ssems[u]).wait()

        rec_issue(0, 0)
        rec_issue(1, 1)

        for sw in range(N_SW):
            wb = sw * SW_WINS
            pltpu.sync_copy(src_hbm.at[pl.ds(ebase + sw * SW_EDGES,
                                             SW_EDGES)], srcsw)
            pltpu.sync_copy(dst_hbm.at[pl.ds(ebase + sw * SW_EDGES,
                                             SW_EDGES)], dstsw)

            def addoff(g, _):
                srcsw[pl.ds(g * 16, 16)] = srcsw[pl.ds(g * 16, 16)] + off
                return _
            lax.fori_loop(0, SW_EDGES // 16, addoff, 0)

            def window(j, k, u, first):
                w = wb + k
                rec_b, x_b, upd_b, dstb_b = recs[u], xs[u], upds[u], dstbs[u]
                nu = 1 - u

                if first:
                    @pl.when(j == 0)
                    def _():
                        rec_wait(wb, u)
                        gather_issue(0, u)
                gather_wait(k, u)
                if sw == 0:
                    @pl.when(w >= 2)
                    def _():
                        scatter_wait(u)
                else:
                    scatter_wait(u)

                def bdst(g, _):
                    dstb_b[pl.ds(g * 16, 16)] = dstsw[
                        pl.ds(k * W_WIN + g * 16, 16)]
                    return _
                lax.fori_loop(0, W_WIN // 16, bdst, 0)

                @plsc.parallel_loop(0, W_WIN, 1, unroll=4)
                def edge(i):
                    shv = rec_b[i, pl.ds(64, 16)]
                    sh1 = shv[0]
                    sh2 = shv[1]
                    sh3 = shv[2]
                    for g in range(2):
                        xg = x_b[i, pl.ds(g * 16, 16)]
                        m0 = rec_b[i, pl.ds(g * 16, 16)] * xg
                        t = rec_b[i, pl.ds(CHUNK + g * 16, 16)] * xg
                        upd_b[i, pl.ds(g * 16, 16)] = m0
                        upd_b[i, pl.ds(32 + g * 16, 16)] = t * sh1
                        upd_b[i, pl.ds(64 + g * 16, 16)] = t * sh2
                        upd_b[i, pl.ds(96 + g * 16, 16)] = t * sh3

                pltpu.async_copy(upd_b, acc.at[dstb_b], ssems[u], add=True)

                if sw == N_SW - 1:
                    @pl.when(w < N_WIN - 2)
                    def _():
                        rec_issue(w + 2, u)
                else:
                    rec_issue(w + 2, u)

                def prep(_k):
                    rec_wait(w + 1, nu)
                    gather_issue(_k, nu)
                if u == 0:
                    prep(k + 1)
                else:
                    @pl.when(j < SW_WINS // 2 - 1)
                    def _():
                        prep(k + 1)

            def body2(j, _):
                window(j, 2 * j, 0, True)
                window(j, 2 * j + 1, 1, False)
                return _
            lax.fori_loop(0, SW_WINS // 2, body2, 0)

        scatter_wait(0)
        scatter_wait(1)
        plsc.subcore_barrier()

        pltpu.sync_copy(acc.at[pl.ds(rbase, ROWS_PER_TILE)],
                        agg_hbm.at[pl.ds(chunk * N_NODES + rbase,
                                         ROWS_PER_TILE)])
        @pl.when(si == NSC - 1)
        def _():
            base = NSC * ROWS_PER_TILE
            pltpu.sync_copy(acc.at[pl.ds(base, 16)],
                            agg_hbm.at[pl.ds(chunk * N_NODES + base, 16)])
        plsc.subcore_barrier()


def _sc_aggregate(xflat, src, dst, recf):
    mesh = plsc.VectorSubcoreMesh(core_axis_name="c", subcore_axis_name="s")
    f = functools.partial(
        pl.kernel,
        out_type=jax.ShapeDtypeStruct((N_CHUNKS * N_NODES, 128), jnp.float32),
        mesh=mesh,
        scratch_types=[
            pltpu.VMEM((SW_EDGES,), jnp.int32),
            pltpu.VMEM((SW_EDGES,), jnp.int32),
            pltpu.VMEM((W_WIN, RECU), jnp.float32),
            pltpu.VMEM((W_WIN, RECU), jnp.float32),
            pltpu.VMEM((W_WIN, CHUNK), jnp.float32),
            pltpu.VMEM((W_WIN, CHUNK), jnp.float32),
            pltpu.VMEM((W_WIN, 128), jnp.float32),
            pltpu.VMEM((W_WIN, 128), jnp.float32),
            pltpu.VMEM((W_WIN,), jnp.int32),
            pltpu.VMEM((W_WIN,), jnp.int32),
            pltpu.VMEM((16, 128), jnp.float32),
            pltpu.VMEM_SHARED((N_NODES, 128), jnp.float32),
            pltpu.SemaphoreType.DMA,
            pltpu.SemaphoreType.DMA,
            pltpu.SemaphoreType.DMA,
            pltpu.SemaphoreType.DMA,
            pltpu.SemaphoreType.DMA,
            pltpu.SemaphoreType.DMA,
        ],
        compiler_params=pltpu.CompilerParams(use_tc_tiling_on_sc=False),
    )(_sc_body)
    return f(xflat, src, dst, recf)


def _out_body(agg_ref, w0_ref, b0_ref, w1_ref, perm_ref, out_ref):
    agg0 = jnp.concatenate([agg_ref[k, :, 0:CHUNK] for k in range(4)], axis=1)
    out0 = jnp.dot(agg0, w0_ref[...],
                   preferred_element_type=jnp.float32) + b0_ref[...]
    outs = []
    for m in range(3):
        a1m = jnp.concatenate(
            [agg_ref[k, :, (m + 1) * CHUNK:(m + 2) * CHUNK] for k in range(4)],
            axis=1)
        outs.append(jnp.dot(a1m, w1_ref[...],
                            preferred_element_type=jnp.float32))
    cat = jnp.concatenate(outs, axis=1)
    out1 = jnp.dot(cat, perm_ref[...],
                   preferred_element_type=jnp.float32,
                   precision=lax.Precision.HIGHEST)
    out_ref[...] = jnp.concatenate([out0, out1], axis=1)


def _out_linear(agg4, w0n, b0, w1n, perm):
    bn = 1000
    grid = (N_NODES // bn,)
    return pl.pallas_call(
        _out_body,
        grid=grid,
        in_specs=[
            pl.BlockSpec((N_CHUNKS, bn, 128), lambda i: (0, i, 0)),
            pl.BlockSpec((128, 128), lambda i: (0, 0)),
            pl.BlockSpec((128,), lambda i: (0,)),
            pl.BlockSpec((128, 64), lambda i: (0, 0)),
            pl.BlockSpec((192, 192), lambda i: (0, 0)),
        ],
        out_specs=pl.BlockSpec((bn, 320), lambda i: (i, 0)),
        out_shape=jax.ShapeDtypeStruct((N_NODES, 320), jnp.float32),
    )(agg4, w0n, b0, w1n, perm)


def kernel(x, edge_index, edge_attr, edge_sh, W_fc1, W_fc2, W0, b0, W1):
    import numpy as np
    src = edge_index[0]
    dst = edge_index[1]
    wf1 = W_fc1 / math.sqrt(float(D_EDGE))
    wf2 = W_fc2 / math.sqrt(float(HIDDEN))

    rec4 = _edge_mlp(edge_attr, edge_sh, wf1, wf2)
    recf = rec4.reshape(N_CHUNKS * N_EDGES, RECW)

    xflat = x.reshape(N_NODES, N_CHUNKS, CHUNK).transpose(1, 0, 2) \
             .reshape(N_CHUNKS * N_NODES, CHUNK)

    aggf = _sc_aggregate(xflat, src, dst, recf)
    agg4 = aggf.reshape(N_CHUNKS, N_NODES, 128)

    w0n = W0 / (math.sqrt(128.0) * DENOM)
    w1n = W1 / (math.sqrt(128.0) * DENOM)
    pm = np.zeros((192, 192), np.float32)
    i_src = np.arange(192)
    pm[i_src, 3 * (i_src % 64) + i_src // 64] = 1.0
    perm = jnp.asarray(pm)

    return _out_linear(agg4, w0n, b0, w1n, perm)
